# baseline (speedup 1.0000x reference)
import functools

import jax
import jax.numpy as jnp
from jax.experimental import pallas as pl
from jax.experimental.pallas import tpu as pltpu

_WINDOW = 64
_TOP_K = 8


def _router_kernel(hid_ref, w_hbm_ref, b_ref, comb_ref, idx_ref, wts_ref,
                   x_vmem, w_vmem, sem_x, sem_w, sem_out):
    B = comb_ref.shape[0]
    S = hid_ref.shape[1]
    cps_in = [
        pltpu.make_async_copy(
            hid_ref.at[bb, S - _WINDOW:, :], x_vmem.at[bb], sem_x.at[bb])
        for bb in range(B)
    ]
    cp_w = pltpu.make_async_copy(w_hbm_ref, w_vmem, sem_w)
    for cp in cps_in:
        cp.start()
    cp_w.start()
    cps_out = []
    for bb, cp in enumerate(cps_in):
        cp.wait()
        cp_out = pltpu.make_async_copy(
            x_vmem.at[bb], comb_ref.at[bb], sem_out.at[bb])
        cp_out.start()
        cps_out.append(cp_out)
    pooled = jnp.mean(x_vmem[...], axis=1)
    cp_w.wait()
    logits = jax.lax.dot_general(
        pooled, w_vmem[...],
        dimension_numbers=(((1,), (1,)), ((), ())),
        preferred_element_type=jnp.float32,
    ) + b_ref[...]
    cols = jax.lax.broadcasted_iota(jnp.int32, logits.shape, 1)
    neg = jnp.float32(-3.0e38)
    work = logits
    vals = []
    idxs = []
    for _ in range(_TOP_K):
        m = jnp.max(work, axis=1, keepdims=True)
        i = jnp.argmax(work, axis=1)[:, None]
        vals.append(m)
        idxs.append(i)
        work = jnp.where(cols == i, neg, work)
    v = jnp.concatenate(vals, axis=1)
    e = jnp.exp(v - v[:, :1])
    wts_ref[...] = e / jnp.sum(e, axis=1, keepdims=True)
    idx_ref[...] = jnp.concatenate(idxs, axis=1).astype(jnp.int32)
    for cp in cps_out:
        cp.wait()


@functools.partial(jax.jit, static_argnums=())
def kernel(hidden_states, cached_states, W, b):
    del cached_states
    B, S, H = hidden_states.shape
    C = W.shape[0]
    out = pl.pallas_call(
        _router_kernel,
        grid=(1,),
        in_specs=[
            pl.BlockSpec(memory_space=pl.ANY),
            pl.BlockSpec(memory_space=pl.ANY),
            pl.BlockSpec((1, C), lambda i: (0, 0)),
        ],
        out_specs=[
            pl.BlockSpec(memory_space=pl.ANY),
            pl.BlockSpec((B, _TOP_K), lambda i: (0, 0)),
            pl.BlockSpec((B, _TOP_K), lambda i: (0, 0)),
        ],
        out_shape=[
            jax.ShapeDtypeStruct((B, _WINDOW, H), jnp.float32),
            jax.ShapeDtypeStruct((B, _TOP_K), jnp.int32),
            jax.ShapeDtypeStruct((B, _TOP_K), jnp.float32),
        ],
        scratch_shapes=[
            pltpu.VMEM((B, _WINDOW, H), jnp.float32),
            pltpu.VMEM((C, H), jnp.float32),
            pltpu.SemaphoreType.DMA((B,)),
            pltpu.SemaphoreType.DMA,
            pltpu.SemaphoreType.DMA((B,)),
        ],
    )(hidden_states, W, b.reshape(1, C))
    combined, top_k_indices, top_k_weights = out
    return (top_k_indices, top_k_weights, combined)


# speedup vs baseline: 1.0151x; 1.0151x over previous
import functools

import jax
import jax.numpy as jnp
from jax.experimental import pallas as pl
from jax.experimental.pallas import tpu as pltpu

_WINDOW = 64
_TOP_K = 8


def _router_kernel(hid_ref, w_hbm_ref, b_ref, comb_ref, idx_ref, wts_ref,
                   x_vmem, w_vmem, idx_vmem, wts_vmem,
                   sem_x, sem_w, sem_out, sem_small):
    B = comb_ref.shape[0]
    S = hid_ref.shape[1]
    cps_in = [
        pltpu.make_async_copy(
            hid_ref.at[bb, S - _WINDOW:, :], x_vmem.at[bb], sem_x.at[bb])
        for bb in range(B)
    ]
    cp_w = pltpu.make_async_copy(w_hbm_ref, w_vmem, sem_w)
    for cp in cps_in:
        cp.start()
    cp_w.start()
    cps_out = []
    for bb, cp in enumerate(cps_in):
        cp.wait()
        cp_out = pltpu.make_async_copy(
            x_vmem.at[bb], comb_ref.at[bb], sem_out.at[bb])
        cp_out.start()
        cps_out.append(cp_out)
    pooled = jnp.mean(x_vmem[...], axis=1)
    cp_w.wait()
    logits = jax.lax.dot_general(
        pooled, w_vmem[...],
        dimension_numbers=(((1,), (1,)), ((), ())),
        preferred_element_type=jnp.float32,
    ) + b_ref[...]
    cols = jax.lax.broadcasted_iota(jnp.int32, logits.shape, 1)
    neg = jnp.float32(-3.0e38)
    work = logits
    vals = []
    idxs = []
    for _ in range(_TOP_K):
        m = jnp.max(work, axis=1, keepdims=True)
        i = jnp.argmax(work, axis=1)[:, None]
        vals.append(m)
        idxs.append(i)
        work = jnp.where(cols == i, neg, work)
    v = jnp.concatenate(vals, axis=1)
    e = jnp.exp(v - v[:, :1])
    wts_vmem[...] = e / jnp.sum(e, axis=1, keepdims=True)
    idx_vmem[...] = jnp.concatenate(idxs, axis=1).astype(jnp.int32)
    cp_idx = pltpu.make_async_copy(idx_vmem, idx_ref, sem_small.at[0])
    cp_wts = pltpu.make_async_copy(wts_vmem, wts_ref, sem_small.at[1])
    cp_idx.start()
    cp_wts.start()
    cp_idx.wait()
    cp_wts.wait()
    for cp in cps_out:
        cp.wait()


@functools.partial(jax.jit, static_argnums=())
def kernel(hidden_states, cached_states, W, b):
    del cached_states
    B, S, H = hidden_states.shape
    C = W.shape[0]
    out = pl.pallas_call(
        _router_kernel,
        grid=(1,),
        in_specs=[
            pl.BlockSpec(memory_space=pl.ANY),
            pl.BlockSpec(memory_space=pl.ANY),
            pl.BlockSpec((1, C), lambda i: (0, 0)),
        ],
        out_specs=[
            pl.BlockSpec(memory_space=pl.ANY),
            pl.BlockSpec(memory_space=pl.ANY),
            pl.BlockSpec(memory_space=pl.ANY),
        ],
        out_shape=[
            jax.ShapeDtypeStruct((B, _WINDOW, H), jnp.float32),
            jax.ShapeDtypeStruct((B, _TOP_K), jnp.int32),
            jax.ShapeDtypeStruct((B, _TOP_K), jnp.float32),
        ],
        scratch_shapes=[
            pltpu.VMEM((B, _WINDOW, H), jnp.float32),
            pltpu.VMEM((C, H), jnp.float32),
            pltpu.VMEM((B, _TOP_K), jnp.int32),
            pltpu.VMEM((B, _TOP_K), jnp.float32),
            pltpu.SemaphoreType.DMA((B,)),
            pltpu.SemaphoreType.DMA,
            pltpu.SemaphoreType.DMA((B,)),
            pltpu.SemaphoreType.DMA((2,)),
        ],
    )(hidden_states, W, b.reshape(1, C))
    combined, top_k_indices, top_k_weights = out
    return (top_k_indices, top_k_weights, combined)


# speedup vs baseline: 1.1371x; 1.1202x over previous
import functools

import jax
import jax.numpy as jnp
from jax.experimental import pallas as pl
from jax.experimental.pallas import tpu as pltpu

_WINDOW = 64
_TOP_K = 8


def _router_kernel(hid_ref, w_hbm_ref, b_ref, comb_ref, idx_ref, wts_ref,
                   x_vmem, w_vmem, b_vmem, idx_vmem, wts_vmem,
                   sem_x, sem_w, sem_out, sem_small):
    B = comb_ref.shape[0]
    S = hid_ref.shape[1]
    cps_in = [
        pltpu.make_async_copy(
            hid_ref.at[bb, S - _WINDOW:, :], x_vmem.at[bb], sem_x.at[bb])
        for bb in range(B)
    ]
    cp_w = pltpu.make_async_copy(w_hbm_ref, w_vmem, sem_w)
    cp_b = pltpu.make_async_copy(b_ref, b_vmem, sem_small.at[0])
    for cp in cps_in:
        cp.start()
    cp_w.start()
    cp_b.start()
    cps_out = []
    for bb, cp in enumerate(cps_in):
        cp.wait()
        cp_out = pltpu.make_async_copy(
            x_vmem.at[bb], comb_ref.at[bb], sem_out.at[bb])
        cp_out.start()
        cps_out.append(cp_out)
    pooled = jnp.mean(x_vmem[...], axis=1)
    cp_w.wait()
    cp_b.wait()
    logits = jax.lax.dot_general(
        pooled, w_vmem[...],
        dimension_numbers=(((1,), (1,)), ((), ())),
        preferred_element_type=jnp.float32,
    ) + b_vmem[...]
    cols = jax.lax.broadcasted_iota(jnp.int32, logits.shape, 1)
    neg = jnp.float32(-3.0e38)
    work = logits
    vals = []
    idxs = []
    for _ in range(_TOP_K):
        m = jnp.max(work, axis=1, keepdims=True)
        i = jnp.argmax(work, axis=1)[:, None]
        vals.append(m)
        idxs.append(i)
        work = jnp.where(cols == i, neg, work)
    v = jnp.concatenate(vals, axis=1)
    e = jnp.exp(v - v[:, :1])
    wts_vmem[...] = e / jnp.sum(e, axis=1, keepdims=True)
    idx_vmem[...] = jnp.concatenate(idxs, axis=1).astype(jnp.int32)
    cp_idx = pltpu.make_async_copy(idx_vmem, idx_ref, sem_small.at[0])
    cp_wts = pltpu.make_async_copy(wts_vmem, wts_ref, sem_small.at[1])
    cp_idx.start()
    cp_wts.start()
    cp_idx.wait()
    cp_wts.wait()
    for cp in cps_out:
        cp.wait()


@functools.partial(jax.jit, static_argnums=())
def kernel(hidden_states, cached_states, W, b):
    del cached_states
    B, S, H = hidden_states.shape
    C = W.shape[0]
    out = pl.pallas_call(
        _router_kernel,
        grid=(1,),
        in_specs=[
            pl.BlockSpec(memory_space=pl.ANY),
            pl.BlockSpec(memory_space=pl.ANY),
            pl.BlockSpec(memory_space=pl.ANY),
        ],
        out_specs=[
            pl.BlockSpec(memory_space=pl.ANY),
            pl.BlockSpec(memory_space=pl.ANY),
            pl.BlockSpec(memory_space=pl.ANY),
        ],
        out_shape=[
            jax.ShapeDtypeStruct((B, _WINDOW, H), jnp.float32),
            jax.ShapeDtypeStruct((B, _TOP_K), jnp.int32),
            jax.ShapeDtypeStruct((B, _TOP_K), jnp.float32),
        ],
        scratch_shapes=[
            pltpu.VMEM((B, _WINDOW, H), jnp.float32),
            pltpu.VMEM((C, H), jnp.float32),
            pltpu.VMEM((1, C), jnp.float32),
            pltpu.VMEM((B, _TOP_K), jnp.int32),
            pltpu.VMEM((B, _TOP_K), jnp.float32),
            pltpu.SemaphoreType.DMA((B,)),
            pltpu.SemaphoreType.DMA,
            pltpu.SemaphoreType.DMA((B,)),
            pltpu.SemaphoreType.DMA((2,)),
        ],
    )(hidden_states, W, b.reshape(1, C))
    combined, top_k_indices, top_k_weights = out
    return (top_k_indices, top_k_weights, combined)
